# baseline (speedup 1.0000x reference)
import functools

import jax
import jax.numpy as jnp
from jax.experimental import pallas as pl
from jax.experimental.pallas import tpu as pltpu

F = 1024
D = 1024
K = 8
EPS = 1e-3
BB = 512
NH = 1
HB = BB // NH


def _dot_t(a, b):
    return jax.lax.dot_general(
        a, b, (((1,), (1,)), ((), ())), preferred_element_type=jnp.float32)


def _pursuit_body(x_ref, y_ref, xs_ref, ys_ref,
                  wout_ref, iout_ref, xrec_ref, yrec_ref, loss_ref):
    xs = xs_ref[...]
    ys = ys_ref[...]
    xsb = xs.astype(jnp.bfloat16)
    col = jax.lax.broadcasted_iota(jnp.int32, (HB, F), 1)
    xh = [x_ref[pl.ds(h * HB, HB), :] for h in range(NH)]

    def iteration(w, x):
        residual = x - jnp.dot(w.astype(jnp.bfloat16), xsb,
                               preferred_element_type=jnp.float32)
        ip = _dot_t(residual.astype(jnp.bfloat16), xsb)
        idx = jnp.argmax(ip, axis=1)[:, None]
        mask = (w != 0.0) | (col == idx)
        grad = jnp.where(mask, ip, 0.0)
        c = jnp.dot(grad.astype(jnp.bfloat16), xsb,
                    preferred_element_type=jnp.float32)
        num = jnp.sum(c * residual, axis=1, keepdims=True)
        den = jnp.sum(c * c, axis=1, keepdims=True)
        step = num / jnp.maximum(den, EPS)
        return jax.nn.relu(w + step * grad), idx

    ws = [jnp.zeros((HB, F), jnp.float32) for _ in range(NH)]
    sel = [[] for _ in range(NH)]
    for t in range(K):
        for h in range(NH):
            ws[h], idx = iteration(ws[h], xh[h])
            sel[h].append(idx)

    for h in range(NH):
        w = ws[h]
        rows = pl.ds(h * HB, HB)
        xrec_ref[rows, :] = jnp.dot(w, xs, preferred_element_type=jnp.float32)
        yrec = jnp.dot(w, ys, preferred_element_type=jnp.float32)
        yrec_ref[rows, :] = yrec
        dy = yrec - y_ref[rows, :]
        loss_ref[rows, :] = jnp.sum(dy * dy, axis=1, keepdims=True)

        rawV, maskedV, valids = [], [], []
        for t in range(K):
            v_t = jnp.sum(jnp.where(col == sel[h][t], w, 0.0),
                          axis=1, keepdims=True)
            valid_t = v_t > 0.0
            for s in range(t):
                valid_t = valid_t & (sel[h][t] != sel[h][s])
            rawV.append(v_t)
            valids.append(valid_t)
            maskedV.append(jnp.where(valid_t, v_t, -2.0))
        slotV = jnp.concatenate(maskedV, axis=1)
        slotI = jnp.concatenate(sel[h], axis=1)
        padI = jax.lax.broadcasted_iota(jnp.int32, (HB, 2 * K), 1)
        padV = jnp.zeros(padI.shape, jnp.float32)
        for t in range(K):
            padV = jnp.where(valids[t] & (sel[h][t] == padI), -2.0, padV)
        CV = jnp.concatenate([slotV, padV], axis=1)
        CI = jnp.concatenate([slotI, padI], axis=1)
        vals, inds = [], []
        for _ in range(K):
            mx = jnp.max(CV, axis=1, keepdims=True)
            pick = jnp.min(jnp.where(CV == mx, CI, F), axis=1, keepdims=True)
            vals.append(mx)
            inds.append(pick)
            CV = jnp.where((CI == pick) & (CV == mx), -3.0, CV)
        wout_ref[rows, :] = jnp.concatenate(vals, axis=1)
        iout_ref[rows, :] = jnp.concatenate(inds, axis=1)


@jax.jit
def kernel(x, y, xs, ys):
    B = x.shape[0]
    nblk = B // BB
    row_blk = lambda i: (i, 0)
    fixed = lambda i: (0, 0)
    weights, indices, x_rec, y_rec, losses = pl.pallas_call(
        _pursuit_body,
        grid=(nblk,),
        in_specs=[
            pl.BlockSpec((BB, D), row_blk),
            pl.BlockSpec((BB, D), row_blk),
            pl.BlockSpec((F, D), fixed),
            pl.BlockSpec((F, D), fixed),
        ],
        out_specs=[
            pl.BlockSpec((BB, K), row_blk),
            pl.BlockSpec((BB, K), row_blk),
            pl.BlockSpec((BB, D), row_blk),
            pl.BlockSpec((BB, D), row_blk),
            pl.BlockSpec((BB, 1), row_blk),
        ],
        out_shape=[
            jax.ShapeDtypeStruct((B, K), jnp.float32),
            jax.ShapeDtypeStruct((B, K), jnp.int32),
            jax.ShapeDtypeStruct((B, D), jnp.float32),
            jax.ShapeDtypeStruct((B, D), jnp.float32),
            jax.ShapeDtypeStruct((B, 1), jnp.float32),
        ],
    )(x, y, xs, ys)
    return weights, indices, x_rec, y_rec, losses.reshape(B)


# speedup vs baseline: 1.0050x; 1.0050x over previous
import functools

import jax
import jax.numpy as jnp
from jax.experimental import pallas as pl
from jax.experimental.pallas import tpu as pltpu

F = 1024
D = 1024
K = 8
EPS = 1e-3
BB = 512
NH = 1
HB = BB // NH


def _dot_t(a, b):
    return jax.lax.dot_general(
        a, b, (((1,), (1,)), ((), ())), preferred_element_type=jnp.float32)


def _pursuit_body(x_ref, y_ref, xs_ref, ys_ref,
                  wout_ref, iout_ref, xrec_ref, yrec_ref, loss_ref):
    xs = xs_ref[...]
    ys = ys_ref[...]
    xsb = xs.astype(jnp.bfloat16)
    col = jax.lax.broadcasted_iota(jnp.int32, (HB, F), 1)
    xh = [x_ref[pl.ds(h * HB, HB), :] for h in range(NH)]

    def iteration(w, x):
        residual = x - jnp.dot(w.astype(jnp.bfloat16), xsb,
                               preferred_element_type=jnp.float32)
        ip = _dot_t(residual.astype(jnp.bfloat16), xsb)
        idx = jnp.argmax(ip, axis=1)[:, None]
        mask = (w != 0.0) | (col == idx)
        grad = jnp.where(mask, ip, 0.0)
        c = jnp.dot(grad.astype(jnp.bfloat16), xsb,
                    preferred_element_type=jnp.float32)
        num = jnp.sum(c * residual, axis=1, keepdims=True)
        den = jnp.sum(c * c, axis=1, keepdims=True)
        step = num / jnp.maximum(den, EPS)
        return jax.nn.relu(w + step * grad)

    def body(t, ws):
        return tuple(iteration(w, x) for w, x in zip(ws, xh))

    ws = tuple(jnp.zeros((HB, F), jnp.float32) for _ in range(NH))
    for t in range(K):
        ws = body(t, ws)

    for h in range(NH):
        w = ws[h]
        rows = pl.ds(h * HB, HB)
        wb = w.astype(jnp.bfloat16)
        xrec_ref[rows, :] = jnp.dot(wb, xsb, preferred_element_type=jnp.float32)
        yrec = jnp.dot(wb, ys.astype(jnp.bfloat16),
                       preferred_element_type=jnp.float32)
        yrec_ref[rows, :] = yrec
        dy = yrec - y_ref[rows, :]
        loss_ref[rows, :] = jnp.sum(dy * dy, axis=1, keepdims=True)

        vals, inds = [], []
        for _ in range(K):
            mx = jnp.max(w, axis=1, keepdims=True)
            idx = jnp.min(jnp.where(w == mx, col, F), axis=1, keepdims=True)
            vals.append(mx)
            inds.append(idx)
            w = jnp.where(col == idx, -1.0, w)
        wout_ref[rows, :] = jnp.concatenate(vals, axis=1)
        iout_ref[rows, :] = jnp.concatenate(inds, axis=1)


@jax.jit
def kernel(x, y, xs, ys):
    B = x.shape[0]
    nblk = B // BB
    row_blk = lambda i: (i, 0)
    fixed = lambda i: (0, 0)
    weights, indices, x_rec, y_rec, losses = pl.pallas_call(
        _pursuit_body,
        grid=(nblk,),
        in_specs=[
            pl.BlockSpec((BB, D), row_blk),
            pl.BlockSpec((BB, D), row_blk),
            pl.BlockSpec((F, D), fixed),
            pl.BlockSpec((F, D), fixed),
        ],
        out_specs=[
            pl.BlockSpec((BB, K), row_blk),
            pl.BlockSpec((BB, K), row_blk),
            pl.BlockSpec((BB, D), row_blk),
            pl.BlockSpec((BB, D), row_blk),
            pl.BlockSpec((BB, 1), row_blk),
        ],
        out_shape=[
            jax.ShapeDtypeStruct((B, K), jnp.float32),
            jax.ShapeDtypeStruct((B, K), jnp.int32),
            jax.ShapeDtypeStruct((B, D), jnp.float32),
            jax.ShapeDtypeStruct((B, D), jnp.float32),
            jax.ShapeDtypeStruct((B, 1), jnp.float32),
        ],
    )(x, y, xs, ys)
    return weights, indices, x_rec, y_rec, losses.reshape(B)
